# Initial kernel scaffold; baseline (speedup 1.0000x reference)
import jax
import jax.numpy as jnp
from jax.experimental import pallas as pl


def kernel(inputs, targets, alpha):
    raise NotImplementedError("write your pallas kernel here")


# speedup vs baseline: 4.4910x; 4.4910x over previous
import jax
import jax.numpy as jnp
from jax.experimental import pallas as pl
from jax.experimental.pallas import tpu as pltpu

_WINDOW = 100


def _focal_body(alpha_ref, x_ref, t_ref, k_ref,
                num_m_ref, cnt_m_ref, num_a_ref, pos_ref):
    x0 = x_ref[0, 0]
    x1 = x_ref[0, 1]
    x2 = x_ref[0, 2]
    t = t_ref[0]

    m = jnp.maximum(jnp.maximum(x0, x1), x2)
    e0 = jnp.exp(x0 - m)
    e1 = jnp.exp(x1 - m)
    e2 = jnp.exp(x2 - m)
    se = e0 + e1 + e2

    c1 = t == 1
    c2 = t == 2
    xt = jnp.where(c2, x2, jnp.where(c1, x1, x0))
    et = jnp.where(c2, e2, jnp.where(c1, e1, e0))
    ce = jnp.log(se) - (xt - m)
    pt = et * (1.0 / se)

    a0 = alpha_ref[0]
    a1 = alpha_ref[1]
    a2 = alpha_ref[2]
    at = jnp.where(c2, a2, jnp.where(c1, a1, a0))
    om = 1.0 - pt
    focal = at * (om * om) * ce

    posf = jnp.where(t > 0, 1.0, 0.0)
    pb = posf.astype(jnp.bfloat16)
    r_rows = pb.shape[0]
    zrow = jnp.zeros((1, 128), jnp.bfloat16)
    p_prev = jnp.concatenate([zrow, pb[:r_rows - 1]], axis=0)
    p_next = jnp.concatenate([pb[1:], zrow], axis=0)
    cnt = (jnp.dot(p_prev, k_ref[0], preferred_element_type=jnp.float32)
           + jnp.dot(pb, k_ref[1], preferred_element_type=jnp.float32)
           + jnp.dot(p_next, k_ref[2], preferred_element_type=jnp.float32))
    mf = jnp.where(cnt > 0.5, 1.0, 0.0)

    num_m_ref[0] = jnp.sum(focal * mf, axis=0, keepdims=True)
    cnt_m_ref[0] = jnp.sum(mf, axis=0, keepdims=True)
    num_a_ref[0] = jnp.sum(focal, axis=0, keepdims=True)
    pos_ref[0] = jnp.sum(posf, axis=0, keepdims=True)


@jax.jit
def kernel(inputs, targets, alpha):
    B, S, C = inputs.shape
    R = S // 128
    x = jnp.transpose(inputs, (0, 2, 1)).reshape(B, C, R, 128)
    t4 = targets.reshape(B, R, 128)

    q = jax.lax.broadcasted_iota(jnp.int32, (128, 128), 0)
    p = jax.lax.broadcasted_iota(jnp.int32, (128, 128), 1)
    d = q - p
    k_prev = d >= 128 - _WINDOW
    k_cur = jnp.abs(d) <= _WINDOW
    k_next = d <= _WINDOW - 128
    kmats = jnp.stack([k_prev, k_cur, k_next]).astype(jnp.bfloat16)

    outs = pl.pallas_call(
        _focal_body,
        grid=(B,),
        in_specs=[
            pl.BlockSpec(memory_space=pltpu.SMEM),
            pl.BlockSpec((1, C, R, 128), lambda b: (b, 0, 0, 0)),
            pl.BlockSpec((1, R, 128), lambda b: (b, 0, 0)),
            pl.BlockSpec((3, 128, 128), lambda b: (0, 0, 0)),
        ],
        out_specs=[pl.BlockSpec((1, 1, 128), lambda b: (b, 0, 0))] * 4,
        out_shape=[jax.ShapeDtypeStruct((B, 1, 128), jnp.float32)] * 4,
        compiler_params=pltpu.CompilerParams(
            dimension_semantics=("parallel",),
        ),
        name="masked_focal_loss",
    )(alpha, x, t4, kmats)

    num_m, cnt_m, num_a, posc = [o.sum(axis=(1, 2)) for o in outs]
    has_pos = posc > 0
    num = jnp.where(has_pos, num_m, num_a)
    den = jnp.where(has_pos, cnt_m, jnp.float32(S))
    return jnp.sum(num) / jnp.sum(den)
